# Initial kernel scaffold; baseline (speedup 1.0000x reference)
import jax
import jax.numpy as jnp
from jax.experimental import pallas as pl


def kernel(x, position_embeddings):
    raise NotImplementedError("write your pallas kernel here")


# speedup vs baseline: 1.7278x; 1.7278x over previous
import jax
import jax.numpy as jnp
from jax.experimental import pallas as pl


_ROWS = 512


def _add_kernel(x_ref, pos_ref, out_ref):
    out_ref[...] = x_ref[...] + pos_ref[...][None, :, :]


def kernel(x, position_embeddings):
    B, T, C = x.shape
    pos = position_embeddings[:T]
    grid = (T // _ROWS,)
    return pl.pallas_call(
        _add_kernel,
        grid=grid,
        in_specs=[
            pl.BlockSpec((B, _ROWS, C), lambda t: (0, t, 0)),
            pl.BlockSpec((_ROWS, C), lambda t: (t, 0)),
        ],
        out_specs=pl.BlockSpec((B, _ROWS, C), lambda t: (0, t, 0)),
        out_shape=jax.ShapeDtypeStruct((B, T, C), x.dtype),
    )(x, pos)
